# Initial kernel scaffold; baseline (speedup 1.0000x reference)
import jax
import jax.numpy as jnp
from jax.experimental import pallas as pl


def kernel(features, adj_indices, adj_values, kernel, bias, skip_weight):
    raise NotImplementedError("write your pallas kernel here")


# speedup vs baseline: 4.4637x; 4.4637x over previous
import functools

import jax
import jax.numpy as jnp
from jax import lax
from jax.experimental import pallas as pl
from jax.experimental.pallas import tpu as pltpu
from jax.experimental.pallas import tpu_sc as plsc

N_NODES = 10000
N_EDGES = 320000
D = 128

NC = 2
NS = 16
L = 16
NW = NC * NS
EPW = N_EDGES // NW
CHUNK = 80
NCHUNK = EPW // CHUNK
RPT = 624
TAIL = N_NODES - NS * RPT

_SELU_SCALE = 1.0507009873554805
_SELU_ALPHA = 1.6732632423543772


def _sc_agg_body(feat_hbm, rows_hbm, cols_hbm, vals_hbm, zeros_hbm, out_hbm,
                 cols_v, rows_v, vals_v, gath_v, spmem_agg, gsem):
    cid = lax.axis_index("c")
    sid = lax.axis_index("s")
    wid = cid * NS + sid

    zoff = pl.multiple_of(sid * RPT, 8)
    pltpu.sync_copy(zeros_hbm.at[pl.ds(zoff, RPT)],
                    spmem_agg.at[pl.ds(zoff, RPT)])
    @pl.when(sid == NS - 1)
    def _():
        pltpu.sync_copy(zeros_hbm.at[pl.ds(NS * RPT, TAIL)],
                        spmem_agg.at[pl.ds(NS * RPT, TAIL)])
    plsc.subcore_barrier()

    def chunk_body(g, carry):
        ebase = pl.multiple_of(wid * EPW + g * CHUNK, 8)
        pltpu.sync_copy(cols_hbm.at[pl.ds(ebase, CHUNK)], cols_v)
        pltpu.sync_copy(rows_hbm.at[pl.ds(ebase, CHUNK)], rows_v)
        pltpu.sync_copy(vals_hbm.at[pl.ds(ebase, CHUNK)], vals_v)
        pltpu.async_copy(feat_hbm.at[cols_v], gath_v, gsem).wait()

        def grp_body(k, c2):
            vals_grp = vals_v[pl.ds(k * L, L)]
            for t in range(L):
                v = vals_grp[t]
                e = k * L + t
                for j in range(D // L):
                    sl = pl.ds(j * L, L)
                    gath_v[e, sl] = gath_v[e, sl] * v
            return c2
        lax.fori_loop(0, CHUNK // L, grp_body, 0, unroll=False)

        pltpu.sync_copy(gath_v, spmem_agg.at[rows_v], add=True)
        return carry

    lax.fori_loop(0, NCHUNK, chunk_body, 0, unroll=False)
    plsc.subcore_barrier()

    woff = pl.multiple_of(sid * RPT, 8)
    pltpu.sync_copy(spmem_agg.at[pl.ds(woff, RPT)],
                    out_hbm.at[cid, pl.ds(woff, RPT)])
    @pl.when(sid == NS - 1)
    def _():
        pltpu.sync_copy(spmem_agg.at[pl.ds(NS * RPT, TAIL)],
                        out_hbm.at[cid, pl.ds(NS * RPT, TAIL)])


def _sc_aggregate(features, rows, cols, vals, zeros):
    mesh = plsc.VectorSubcoreMesh(core_axis_name="c", subcore_axis_name="s")
    f = pl.kernel(
        _sc_agg_body,
        out_type=jax.ShapeDtypeStruct((NC, N_NODES, D), jnp.float32),
        mesh=mesh,
        scratch_types=[
            pltpu.VMEM((CHUNK,), jnp.int32),
            pltpu.VMEM((CHUNK,), jnp.int32),
            pltpu.VMEM((CHUNK,), jnp.float32),
            pltpu.VMEM((CHUNK, D), jnp.float32),
            pltpu.VMEM_SHARED((N_NODES, D), jnp.float32),
            pltpu.SemaphoreType.DMA,
        ],
    )
    return f(features, rows, cols, vals, zeros)


def _tc_body(f_ref, p_ref, k_ref, b_ref, sw_ref, o_ref):
    h = jnp.dot(f_ref[...], k_ref[...], preferred_element_type=jnp.float32,
                precision=lax.Precision.HIGHEST)
    agg = jnp.dot(p_ref[0] + p_ref[1], k_ref[...],
                  preferred_element_type=jnp.float32,
                  precision=lax.Precision.HIGHEST)
    y = h * sw_ref[...] + agg + b_ref[...]
    o_ref[...] = jnp.where(
        y > 0.0,
        _SELU_SCALE * y,
        (_SELU_SCALE * _SELU_ALPHA) * (jnp.exp(jnp.minimum(y, 0.0)) - 1.0),
    )


def _tc_finish(features, partials, k, bias2, sw2):
    BM = 2000
    return pl.pallas_call(
        _tc_body,
        grid=(N_NODES // BM,),
        in_specs=[
            pl.BlockSpec((BM, D), lambda i: (i, 0)),
            pl.BlockSpec((NC, BM, D), lambda i: (0, i, 0)),
            pl.BlockSpec((D, D), lambda i: (0, 0)),
            pl.BlockSpec((1, D), lambda i: (0, 0)),
            pl.BlockSpec((1, D), lambda i: (0, 0)),
        ],
        out_specs=pl.BlockSpec((BM, D), lambda i: (i, 0)),
        out_shape=jax.ShapeDtypeStruct((N_NODES, D), jnp.float32),
    )(features, partials, k, bias2, sw2)


def kernel(features, adj_indices, adj_values, kernel, bias, skip_weight):
    rows = adj_indices[0]
    cols = adj_indices[1]
    zeros = jnp.zeros((N_NODES, D), jnp.float32)
    partials = _sc_aggregate(features, rows, cols, adj_values, zeros)
    return _tc_finish(features, partials, kernel,
                      bias.reshape(1, D), skip_weight.reshape(1, D))
